# baseline (speedup 1.0000x reference)
import functools

import jax
import jax.numpy as jnp
from jax import lax
from jax.experimental import pallas as pl
from jax.experimental.pallas import tpu as pltpu
from jax.experimental.pallas import tpu_sc as plsc

B = 4096
IN_BITS = 3072
N1 = 2000
N2 = 1000
K = 6
C = 1000
N1P = 2048
N2P = 1024

NH = 2
BH = B // NH

NC = 2
NS = 16
NW = NC * NS
L = 16

BB = 512


def _build_m(mapt_ref, m_scr, n_in, n_out):
    rc = 256
    iota = lax.broadcasted_iota(jnp.int16, (rc, n_out), 0)

    def chunk(i, carry):
        base = (i * rc).astype(jnp.int16)
        acc = jnp.zeros((rc, n_out), jnp.bfloat16)
        for j in range(K):
            row = mapt_ref[j : j + 1, :].astype(jnp.int16) - base
            acc += jnp.where(iota == row, jnp.bfloat16(2.0**j), jnp.bfloat16(0.0))
        m_scr[pl.ds(i * rc, rc), :] = acc
        return carry

    lax.fori_loop(0, n_in // rc, chunk, 0)


def _mm1_body(mapt_ref, x_ref, out_ref, m_scr):
    @pl.when(pl.program_id(0) == 0)
    def _():
        _build_m(mapt_ref, m_scr, IN_BITS, N1P)

    x = x_ref[...].astype(jnp.bfloat16)
    for n0 in range(0, N1P, 512):
        acc = lax.dot_general(
            x, m_scr[:, n0 : n0 + 512], (((1,), (0,)), ((), ())),
            preferred_element_type=jnp.float32)
        out_ref[:, n0 : n0 + 512] = acc.astype(jnp.int32)


def _mm1(m1t, x_bits, h):
    return pl.pallas_call(
        _mm1_body,
        grid=(BH // BB,),
        in_specs=[
            pl.BlockSpec((K, N1P), lambda b: (0, 0)),
            pl.BlockSpec((BB, IN_BITS), lambda b, h=h: (b + h * (BH // BB), 0)),
        ],
        out_specs=pl.BlockSpec((BB, N1P), lambda b: (b, 0)),
        out_shape=jax.ShapeDtypeStruct((BH, N1P), jnp.int32),
        scratch_shapes=[pltpu.VMEM((IN_BITS, N1P), jnp.bfloat16)],
    )(m1t, x_bits)


def _mm2_body(mapt_ref, bits_ref, out_ref, m_scr):
    @pl.when(pl.program_id(0) == 0)
    def _():
        _build_m(mapt_ref, m_scr, N1P, N2P)

    bb = bits_ref[...].astype(jnp.bfloat16)
    acc = lax.dot_general(
        bb, m_scr[...], (((1,), (0,)), ((), ())),
        preferred_element_type=jnp.float32)
    out_ref[...] = acc.astype(jnp.int32)


def _mm2(m2t, bits1):
    return pl.pallas_call(
        _mm2_body,
        grid=(BH // BB,),
        in_specs=[
            pl.BlockSpec((K, N2P), lambda b: (0, 0)),
            pl.BlockSpec((BB, N1P), lambda b: (b, 0)),
        ],
        out_specs=pl.BlockSpec((BB, N2P), lambda b: (b, 0)),
        out_shape=jax.ShapeDtypeStruct((BH, N2P), jnp.int32),
        scratch_shapes=[pltpu.VMEM((N1P, N2P), jnp.bfloat16)],
    )(m2t, bits1)


def _mm3_body(h_ref, w_ref, out_ref):
    out_ref[...] = lax.dot_general(
        h_ref[...].astype(jnp.bfloat16), w_ref[...], (((1,), (0,)), ((), ())),
        preferred_element_type=jnp.float32)


def _mm3(h2, w_t):
    return pl.pallas_call(
        _mm3_body,
        grid=(BH // BB,),
        in_specs=[
            pl.BlockSpec((BB, N2P), lambda b: (b, 0)),
            pl.BlockSpec((N2P, C), lambda b: (0, 0)),
        ],
        out_specs=pl.BlockSpec((BB, C), lambda b: (b, 0)),
        out_shape=jax.ShapeDtypeStruct((BH, C), jnp.float32),
    )(h2, w_t)


CHUNK = 32768
UNROLL = 8


def _make_sc_lut(n_rows, sigmoid):
    rpw = 128
    col_shards = n_rows // rpw
    b_shards = NW // col_shards
    bspan = BH // b_shards
    rb = CHUNK // rpw
    n_chunks = bspan // rb
    gpr = rpw // L
    mesh = plsc.VectorSubcoreMesh(core_axis_name="c", subcore_axis_name="s")

    @functools.partial(
        pl.kernel,
        out_type=jax.ShapeDtypeStruct((BH, n_rows), jnp.float32),
        mesh=mesh,
        scratch_types=[
            pltpu.VMEM((rpw * 64,), jnp.float32),
            pltpu.VMEM((rb, rpw), jnp.int32),
            pltpu.VMEM((rb, rpw), jnp.float32),
        ],
        compiler_params=pltpu.CompilerParams(needs_layout_passes=False),
    )
    def k(addr_hbm, luts_hbm, out_hbm, luts_v, addr_v, out_v):
        wid = lax.axis_index("s") * NC + lax.axis_index("c")
        col0 = (wid % col_shards) * rpw
        row_base = (wid // col_shards) * bspan
        pltpu.sync_copy(luts_hbm.at[pl.ds(col0 * 64, rpw * 64)], luts_v)
        lane64 = jnp.arange(L, dtype=jnp.int32) * 64
        for c in range(n_chunks):
            pltpu.sync_copy(
                addr_hbm.at[pl.ds(row_base + c * rb, rb), pl.ds(col0, rpw)],
                addr_v)

            gpr_shift = gpr.bit_length() - 1

            @plsc.parallel_loop(0, rb * gpr, 1, unroll=UNROLL)
            def body(gg):
                r = gg >> gpr_shift
                g = gg & (gpr - 1)
                lut0 = g * L
                a = addr_v[r, pl.ds(lut0, L)]
                idx = a + (lut0 << 6) + lane64
                v = plsc.load_gather(luts_v, [idx])
                if sigmoid:
                    o = 1.0 / (1.0 + jnp.exp(-v))
                else:
                    o = jnp.where(v >= 0.0, 1.0, 0.0).astype(jnp.float32)
                out_v[r, pl.ds(lut0, L)] = o

            pltpu.sync_copy(
                out_v,
                out_hbm.at[pl.ds(row_base + c * rb, rb), pl.ds(col0, rpw)])

    return k


_sc_lut1 = _make_sc_lut(N1P, sigmoid=False)
_sc_lut2 = _make_sc_lut(N2P, sigmoid=True)


def kernel(x_bits, luts1, luts2, W, mapping1, mapping2):
    m1t = jnp.pad(mapping1, ((0, N1P - N1), (0, 0))).T
    m2t = jnp.pad(mapping2, ((0, N2P - N2), (0, 0))).T
    l1p = jnp.pad(luts1, ((0, N1P - N1), (0, 0))).reshape(-1)
    l2p = jnp.pad(luts2, ((0, N2P - N2), (0, 0))).reshape(-1)
    w_t = jnp.pad(W, ((0, 0), (0, N2P - N2))).T.astype(jnp.bfloat16)

    halves = []
    for h in range(NH):
        a1 = _mm1(m1t, x_bits, h)
        bits1 = _sc_lut1(a1, l1p)
        a2 = _mm2(m2t, bits1)
        h2 = _sc_lut2(a2, l2p)
        halves.append(_mm3(h2, w_t))
    return jnp.concatenate(halves, axis=0)


# speedup vs baseline: 1.0236x; 1.0236x over previous
import functools

import jax
import jax.numpy as jnp
from jax import lax
from jax.experimental import pallas as pl
from jax.experimental.pallas import tpu as pltpu
from jax.experimental.pallas import tpu_sc as plsc

B = 4096
IN_BITS = 3072
N1 = 2000
N2 = 1000
K = 6
C = 1000
N1P = 2048
N2P = 1024

NH = 2
BH = B // NH

NC = 2
NS = 16
NW = NC * NS
L = 16

BB = 512


def _build_m(mapt_ref, m_scr, n_in, n_out):
    rc = 256
    iota = lax.broadcasted_iota(jnp.int16, (rc, n_out), 0)

    def chunk(i, carry):
        base = (i * rc).astype(jnp.int16)
        acc = jnp.zeros((rc, n_out), jnp.bfloat16)
        for j in range(K):
            row = mapt_ref[j : j + 1, :].astype(jnp.int16) - base
            acc += jnp.where(iota == row, jnp.bfloat16(2.0**j), jnp.bfloat16(0.0))
        m_scr[pl.ds(i * rc, rc), :] = acc
        return carry

    lax.fori_loop(0, n_in // rc, chunk, 0)


def _mm1_body(mapt_ref, x_ref, out_ref, m_scr):
    @pl.when(pl.program_id(0) == 0)
    def _():
        _build_m(mapt_ref, m_scr, IN_BITS, N1P)

    x = x_ref[...].astype(jnp.bfloat16)
    for n0 in range(0, N1P, 512):
        acc = lax.dot_general(
            x, m_scr[:, n0 : n0 + 512], (((1,), (0,)), ((), ())),
            preferred_element_type=jnp.float32)
        out_ref[:, n0 : n0 + 512] = acc.astype(jnp.int32)


def _mm1(m1t, x_bits, h):
    return pl.pallas_call(
        _mm1_body,
        grid=(BH // BB,),
        in_specs=[
            pl.BlockSpec((K, N1P), lambda b: (0, 0)),
            pl.BlockSpec((BB, IN_BITS), lambda b, h=h: (b + h * (BH // BB), 0)),
        ],
        out_specs=pl.BlockSpec((BB, N1P), lambda b: (b, 0)),
        out_shape=jax.ShapeDtypeStruct((BH, N1P), jnp.int32),
        scratch_shapes=[pltpu.VMEM((IN_BITS, N1P), jnp.bfloat16)],
    )(m1t, x_bits)


def _mm2_body(mapt_ref, bits_ref, out_ref, m_scr):
    @pl.when(pl.program_id(0) == 0)
    def _():
        _build_m(mapt_ref, m_scr, N1P, N2P)

    bb = bits_ref[...].astype(jnp.bfloat16)
    acc = lax.dot_general(
        bb, m_scr[...], (((1,), (0,)), ((), ())),
        preferred_element_type=jnp.float32)
    out_ref[...] = acc.astype(jnp.int32)


def _mm2(m2t, bits1):
    return pl.pallas_call(
        _mm2_body,
        grid=(BH // BB,),
        in_specs=[
            pl.BlockSpec((K, N2P), lambda b: (0, 0)),
            pl.BlockSpec((BB, N1P), lambda b: (b, 0)),
        ],
        out_specs=pl.BlockSpec((BB, N2P), lambda b: (b, 0)),
        out_shape=jax.ShapeDtypeStruct((BH, N2P), jnp.int32),
        scratch_shapes=[pltpu.VMEM((N1P, N2P), jnp.bfloat16)],
    )(m2t, bits1)


def _mm3_body(ha_ref, hb_ref, w_ref, out_ref):
    hblocks = BH // BB
    pid = pl.program_id(0)

    @pl.when(pid < hblocks)
    def _():
        out_ref[...] = lax.dot_general(
            ha_ref[...].astype(jnp.bfloat16), w_ref[...],
            (((1,), (0,)), ((), ())), preferred_element_type=jnp.float32)

    @pl.when(pid >= hblocks)
    def _():
        out_ref[...] = lax.dot_general(
            hb_ref[...].astype(jnp.bfloat16), w_ref[...],
            (((1,), (0,)), ((), ())), preferred_element_type=jnp.float32)


def _mm3(h2a, h2b, w_t):
    hblocks = BH // BB
    return pl.pallas_call(
        _mm3_body,
        grid=(B // BB,),
        in_specs=[
            pl.BlockSpec((BB, N2P), lambda b: (jnp.minimum(b, hblocks - 1), 0)),
            pl.BlockSpec((BB, N2P), lambda b: (jnp.maximum(b - hblocks, 0), 0)),
            pl.BlockSpec((N2P, C), lambda b: (0, 0)),
        ],
        out_specs=pl.BlockSpec((BB, C), lambda b: (b, 0)),
        out_shape=jax.ShapeDtypeStruct((B, C), jnp.float32),
    )(h2a, h2b, w_t)


CHUNK = 32768
UNROLL = 8


def _make_sc_lut(n_rows, sigmoid):
    rpw = 128
    col_shards = n_rows // rpw
    b_shards = NW // col_shards
    bspan = BH // b_shards
    rb = CHUNK // rpw
    n_chunks = bspan // rb
    mesh = plsc.VectorSubcoreMesh(core_axis_name="c", subcore_axis_name="s")
    out_dtype = jnp.float32
    in_dtype = jnp.int32

    @functools.partial(
        pl.kernel,
        out_type=jax.ShapeDtypeStruct((BH, n_rows), out_dtype),
        mesh=mesh,
        scratch_types=[
            pltpu.VMEM((rpw * 64,), jnp.float32),
            pltpu.VMEM((rb, rpw), in_dtype),
            pltpu.VMEM((rb, rpw), out_dtype),
        ],
        compiler_params=pltpu.CompilerParams(needs_layout_passes=False),
    )
    def k(addr_hbm, luts_hbm, out_hbm, luts_v, addr_v, out_v):
        wid = lax.axis_index("s") * NC + lax.axis_index("c")
        col0 = (wid % col_shards) * rpw
        row_base = (wid // col_shards) * bspan
        pltpu.sync_copy(luts_hbm.at[pl.ds(col0 * 64, rpw * 64)], luts_v)
        lane64 = jnp.arange(L, dtype=jnp.int32) * 64
        gpr = rpw // L
        gpr_shift = gpr.bit_length() - 1
        for c in range(n_chunks):
            pltpu.sync_copy(
                addr_hbm.at[pl.ds(row_base + c * rb, rb), pl.ds(col0, rpw)],
                addr_v)

            @plsc.parallel_loop(0, rb * gpr, 1, unroll=UNROLL)
            def body(gg):
                r = gg >> gpr_shift
                lut0 = (gg & (gpr - 1)) * L
                a = addr_v[r, pl.ds(lut0, L)]
                idx = a + (lut0 << 6) + lane64
                v = plsc.load_gather(luts_v, [idx])
                if sigmoid:
                    o = 1.0 / (1.0 + jnp.exp(-v))
                else:
                    o = jnp.where(v >= 0.0, 1.0, 0.0).astype(jnp.float32)
                out_v[r, pl.ds(lut0, L)] = o

            pltpu.sync_copy(
                out_v,
                out_hbm.at[pl.ds(row_base + c * rb, rb), pl.ds(col0, rpw)])

    return k


_sc_lut1 = _make_sc_lut(N1P, sigmoid=False)
_sc_lut2 = _make_sc_lut(N2P, sigmoid=True)


def kernel(x_bits, luts1, luts2, W, mapping1, mapping2):
    m1t = jnp.pad(mapping1, ((0, N1P - N1), (0, 0))).T
    m2t = jnp.pad(mapping2, ((0, N2P - N2), (0, 0))).T
    l1p = jnp.pad(luts1, ((0, N1P - N1), (0, 0))).reshape(-1)
    l2p = jnp.pad(luts2, ((0, N2P - N2), (0, 0))).reshape(-1)
    w_t = jnp.pad(W, ((0, 0), (0, N2P - N2))).T.astype(jnp.bfloat16)

    h2s = []
    for h in range(NH):
        a1 = _mm1(m1t, x_bits, h)
        bits1 = _sc_lut1(a1, l1p)
        a2 = _mm2(m2t, bits1)
        h2s.append(_sc_lut2(a2, l2p))
    return _mm3(h2s[0], h2s[1], w_t)


# speedup vs baseline: 1.1340x; 1.1078x over previous
import functools

import jax
import jax.numpy as jnp
from jax import lax
from jax.experimental import pallas as pl
from jax.experimental.pallas import tpu as pltpu
from jax.experimental.pallas import tpu_sc as plsc

B = 4096
IN_BITS = 3072
N1 = 2000
N2 = 1000
K = 6
C = 1000
N1P = 2048
N2P = 1024

NH = 2
BH = B // NH

NC = 2
NS = 16
NW = NC * NS
L = 16

BB = 512


def _build_m(mapt_ref, m_scr, n_in, n_out):
    rc = 256
    iota = lax.broadcasted_iota(jnp.int16, (rc, n_out), 0)

    def chunk(i, carry):
        base = (i * rc).astype(jnp.int16)
        acc = jnp.zeros((rc, n_out), jnp.bfloat16)
        for j in range(K):
            row = mapt_ref[j : j + 1, :].astype(jnp.int16) - base
            acc += jnp.where(iota == row, jnp.bfloat16(2.0**j), jnp.bfloat16(0.0))
        m_scr[pl.ds(i * rc, rc), :] = acc
        return carry

    lax.fori_loop(0, n_in // rc, chunk, 0)


def _mm1_body(mapt_ref, x_ref, out_ref, m_scr):
    @pl.when(pl.program_id(0) == 0)
    def _():
        _build_m(mapt_ref, m_scr, IN_BITS, N1P)

    x = x_ref[...].astype(jnp.bfloat16)
    for n0 in range(0, N1P, 512):
        acc = lax.dot_general(
            x, m_scr[:, n0 : n0 + 512], (((1,), (0,)), ((), ())),
            preferred_element_type=jnp.float32)
        out_ref[:, n0 : n0 + 512] = acc.astype(jnp.int32)


def _mm1(m1t, x_bits, h):
    return pl.pallas_call(
        _mm1_body,
        grid=(BH // BB,),
        in_specs=[
            pl.BlockSpec((K, N1P), lambda b: (0, 0)),
            pl.BlockSpec((BB, IN_BITS), lambda b, h=h: (b + h * (BH // BB), 0)),
        ],
        out_specs=pl.BlockSpec((BB, N1P), lambda b: (b, 0)),
        out_shape=jax.ShapeDtypeStruct((BH, N1P), jnp.int32),
        scratch_shapes=[pltpu.VMEM((IN_BITS, N1P), jnp.bfloat16)],
    )(m1t, x_bits)


def _mm2_body(mapt_ref, bits_ref, out_ref, m_scr):
    @pl.when(pl.program_id(0) == 0)
    def _():
        _build_m(mapt_ref, m_scr, N1P, N2P)

    bb = bits_ref[...].astype(jnp.bfloat16)
    acc = lax.dot_general(
        bb, m_scr[...], (((1,), (0,)), ((), ())),
        preferred_element_type=jnp.float32)
    out_ref[...] = acc.astype(jnp.int32)


def _mm2(m2t, bits1):
    return pl.pallas_call(
        _mm2_body,
        grid=(BH // BB,),
        in_specs=[
            pl.BlockSpec((K, N2P), lambda b: (0, 0)),
            pl.BlockSpec((BB, N1P), lambda b: (b, 0)),
        ],
        out_specs=pl.BlockSpec((BB, N2P), lambda b: (b, 0)),
        out_shape=jax.ShapeDtypeStruct((BH, N2P), jnp.int32),
        scratch_shapes=[pltpu.VMEM((N1P, N2P), jnp.bfloat16)],
    )(m2t, bits1)


def _mm3_body(ha_ref, hb_ref, w_ref, out_ref):
    hblocks = BH // BB
    pid = pl.program_id(0)

    @pl.when(pid < hblocks)
    def _():
        out_ref[...] = lax.dot_general(
            ha_ref[...].astype(jnp.bfloat16), w_ref[...],
            (((1,), (0,)), ((), ())), preferred_element_type=jnp.float32)

    @pl.when(pid >= hblocks)
    def _():
        out_ref[...] = lax.dot_general(
            hb_ref[...].astype(jnp.bfloat16), w_ref[...],
            (((1,), (0,)), ((), ())), preferred_element_type=jnp.float32)


def _mm3(h2a, h2b, w_t):
    hblocks = BH // BB
    return pl.pallas_call(
        _mm3_body,
        grid=(B // BB,),
        in_specs=[
            pl.BlockSpec((BB, N2P), lambda b: (jnp.minimum(b, hblocks - 1), 0)),
            pl.BlockSpec((BB, N2P), lambda b: (jnp.maximum(b - hblocks, 0), 0)),
            pl.BlockSpec((N2P, C), lambda b: (0, 0)),
        ],
        out_specs=pl.BlockSpec((BB, C), lambda b: (b, 0)),
        out_shape=jax.ShapeDtypeStruct((B, C), jnp.float32),
    )(h2a, h2b, w_t)


CHUNK = 16384
UNROLL = 8


def _make_sc_lut(n_rows, sigmoid):
    rpw = 128
    col_shards = n_rows // rpw
    b_shards = NW // col_shards
    bspan = BH // b_shards
    rb = CHUNK // rpw
    n_chunks = bspan // rb
    mesh = plsc.VectorSubcoreMesh(core_axis_name="c", subcore_axis_name="s")
    out_dtype = jnp.float32
    in_dtype = jnp.int32

    @functools.partial(
        pl.kernel,
        out_type=jax.ShapeDtypeStruct((BH, n_rows), out_dtype),
        mesh=mesh,
        scratch_types=[
            pltpu.VMEM((rpw * 64,), jnp.float32),
            pltpu.VMEM((rb, rpw), in_dtype),
            pltpu.VMEM((rb, rpw), in_dtype),
            pltpu.VMEM((rb, rpw), out_dtype),
            pltpu.VMEM((rb, rpw), out_dtype),
            pltpu.SemaphoreType.DMA,
            pltpu.SemaphoreType.DMA,
            pltpu.SemaphoreType.DMA,
            pltpu.SemaphoreType.DMA,
        ],
        compiler_params=pltpu.CompilerParams(needs_layout_passes=False),
    )
    def k(addr_hbm, luts_hbm, out_hbm, luts_v, a0, a1, o0, o1,
          si0, si1, so0, so1):
        addr_bufs, out_bufs = [a0, a1], [o0, o1]
        sis, sos = [si0, si1], [so0, so1]
        wid = lax.axis_index("s") * NC + lax.axis_index("c")
        col0 = (wid % col_shards) * rpw
        row_base = (wid // col_shards) * bspan
        pltpu.sync_copy(luts_hbm.at[pl.ds(col0 * 64, rpw * 64)], luts_v)
        lane64 = jnp.arange(L, dtype=jnp.int32) * 64
        gpr = rpw // L
        gpr_shift = gpr.bit_length() - 1

        def in_slice(c):
            return addr_hbm.at[pl.ds(row_base + c * rb, rb), pl.ds(col0, rpw)]

        def out_slice(c):
            return out_hbm.at[pl.ds(row_base + c * rb, rb), pl.ds(col0, rpw)]

        in_dmas = {0: pltpu.async_copy(in_slice(0), addr_bufs[0], sis[0])}
        out_dmas = {}
        for c in range(n_chunks):
            if c + 1 < n_chunks:
                in_dmas[c + 1] = pltpu.async_copy(
                    in_slice(c + 1), addr_bufs[(c + 1) % 2], sis[(c + 1) % 2])
            in_dmas[c].wait()
            if c >= 2:
                out_dmas[c - 2].wait()
            addr_v = addr_bufs[c % 2]
            out_v = out_bufs[c % 2]

            @plsc.parallel_loop(0, rb * gpr, 1, unroll=UNROLL)
            def body(gg):
                r = gg >> gpr_shift
                lut0 = (gg & (gpr - 1)) * L
                a = addr_v[r, pl.ds(lut0, L)]
                idx = a + (lut0 << 6) + lane64
                v = plsc.load_gather(luts_v, [idx])
                if sigmoid:
                    o = 1.0 / (1.0 + jnp.exp(-v))
                else:
                    o = jnp.where(v >= 0.0, 1.0, 0.0).astype(jnp.float32)
                out_v[r, pl.ds(lut0, L)] = o

            out_dmas[c] = pltpu.async_copy(out_v, out_slice(c), sos[c % 2])
        for c in range(max(0, n_chunks - 2), n_chunks):
            out_dmas[c].wait()

    return k


_sc_lut1 = _make_sc_lut(N1P, sigmoid=False)
_sc_lut2 = _make_sc_lut(N2P, sigmoid=True)


def kernel(x_bits, luts1, luts2, W, mapping1, mapping2):
    m1t = jnp.pad(mapping1, ((0, N1P - N1), (0, 0))).T
    m2t = jnp.pad(mapping2, ((0, N2P - N2), (0, 0))).T
    l1p = jnp.pad(luts1, ((0, N1P - N1), (0, 0))).reshape(-1)
    l2p = jnp.pad(luts2, ((0, N2P - N2), (0, 0))).reshape(-1)
    w_t = jnp.pad(W, ((0, 0), (0, N2P - N2))).T.astype(jnp.bfloat16)

    h2s = []
    for h in range(NH):
        a1 = _mm1(m1t, x_bits, h)
        bits1 = _sc_lut1(a1, l1p)
        a2 = _mm2(m2t, bits1)
        h2s.append(_sc_lut2(a2, l2p))
    return _mm3(h2s[0], h2s[1], w_t)
